# baseline (speedup 1.0000x reference)
import functools

import jax
import jax.numpy as jnp
import numpy as np
from jax import lax
from jax.experimental import pallas as pl
from jax.experimental.pallas import tpu as pltpu
from jax.experimental.pallas import tpu_sc as plsc

_L = 16
_BITS = [int(np.uint32(1 << b).astype(np.int32)) for b in range(32)]


@functools.lru_cache(maxsize=None)
def _build(flat: int, table: int):
  mesh = plsc.VectorSubcoreMesh(core_axis_name="c", subcore_axis_name="s")
  nc, ns = mesh.num_cores, mesh.num_subcores
  nw = nc * ns
  per_w = flat // nw
  assert per_w * nw == flat
  chunk = 10240
  assert per_w % chunk == 0
  nchunk = per_w // chunk
  assert nchunk >= 2

  nwords = table // 32
  per_sub_ent = table // ns
  per_sub_w = nwords // ns
  wblk = 8192
  nwblk = per_sub_ent // wblk
  gpb = wblk // (32 * _L)
  assert table % 32 == 0 and nwords % ns == 0 and per_sub_ent % wblk == 0

  def body(x_hbm, idx_hbm, w_hbm, out_hbm, idx0, idx1, x0, x1, o0, o1,
           mask_v, word_v, shared, is0, is1, os0, os1, msem):
    s = lax.axis_index("s")
    lanes = lax.iota(jnp.int32, _L)
    w_chunk = o0

    ent_base = pl.multiple_of(s * per_sub_ent, wblk)
    for kb in range(nwblk):
      pltpu.sync_copy(w_hbm.at[pl.ds(ent_base + kb * wblk, wblk)],
                      w_chunk.at[pl.ds(0, wblk)])

      def pack_group(g, _, kb=kb):
        base = lanes * 32 + g * (32 * _L)
        acc = jnp.zeros((_L,), jnp.int32)
        for b in range(32):
          v = plsc.load_gather(w_chunk, [base + b])
          acc = acc | jnp.where(v > 0.5, jnp.int32(_BITS[b]), jnp.int32(0))
        word_v[pl.ds(pl.multiple_of(kb * gpb * _L + g * _L, _L), _L)] = acc
        return _

      lax.fori_loop(0, gpb, pack_group, None)

    pltpu.sync_copy(word_v,
                    shared.at[pl.ds(pl.multiple_of(s * per_sub_w, 8),
                                    per_sub_w)])
    plsc.subcore_barrier()
    pltpu.sync_copy(shared, mask_v)

    wid = s * nc + lax.axis_index("c")
    idxb, xb, ob = [idx0, idx1], [x0, x1], [o0, o1]
    isem, osem = [is0, is1], [os0, os1]

    def cbase(k):
      return pl.multiple_of(wid * per_w + k * chunk, chunk)

    ih, oh = {}, {}

    def start_in(k):
      p = k & 1
      b = cbase(k)
      ih[k] = (
          pltpu.async_copy(idx_hbm.at[pl.ds(b, chunk)], idxb[p], isem[p]),
          pltpu.async_copy(x_hbm.at[pl.ds(b, chunk)], xb[p], isem[p]),
      )

    start_in(0)
    start_in(1)

    for k in range(nchunk):
      p = k & 1
      ha, hb = ih.pop(k)
      ha.wait()
      hb.wait()
      if k >= 2:
        oh.pop(k - 2).wait()
      iv_ref, xv, ov = idxb[p], xb[p], ob[p]

      @plsc.parallel_loop(0, chunk // _L, 1, unroll=4)
      def _(i):
        sl = pl.ds(pl.multiple_of(i * _L, _L), _L)
        iv = iv_ref[sl]
        words = plsc.load_gather(mask_v, [lax.shift_right_logical(iv, 5)])
        t = lax.shift_left(words, jnp.bitwise_and(jnp.bitwise_not(iv), 31))
        m = lax.shift_right_arithmetic(t, 31)
        ov[sl] = plsc.bitcast(plsc.bitcast(xv[sl], jnp.int32) & m,
                              jnp.float32)

      oh[k] = pltpu.async_copy(ov, out_hbm.at[pl.ds(cbase(k), chunk)],
                               osem[p])
      if k + 2 < nchunk:
        start_in(k + 2)

    oh.pop(nchunk - 2).wait()
    oh.pop(nchunk - 1).wait()

  return pl.kernel(
      body,
      out_type=jax.ShapeDtypeStruct((flat,), jnp.float32),
      mesh=mesh,
      compiler_params=pltpu.CompilerParams(needs_layout_passes=False),
      scratch_types=[
          pltpu.VMEM((chunk,), jnp.int32),
          pltpu.VMEM((chunk,), jnp.int32),
          pltpu.VMEM((chunk,), jnp.float32),
          pltpu.VMEM((chunk,), jnp.float32),
          pltpu.VMEM((chunk,), jnp.float32),
          pltpu.VMEM((chunk,), jnp.float32),
          pltpu.VMEM((nwords,), jnp.int32),
          pltpu.VMEM((per_sub_w,), jnp.int32),
          pltpu.VMEM_SHARED((nwords,), jnp.int32),
          pltpu.SemaphoreType.DMA,
          pltpu.SemaphoreType.DMA,
          pltpu.SemaphoreType.DMA,
          pltpu.SemaphoreType.DMA,
          pltpu.SemaphoreType.DMA,
      ],
  )


def kernel(x, idx, W):
  flat = x.size
  xf = x.reshape(flat)
  idxf = idx.reshape(flat).astype(jnp.int32)
  align = 16 * 8192
  table = ((W.shape[0] + align - 1) // align) * align
  if table != W.shape[0]:
    W = jnp.concatenate([W, jnp.zeros((table - W.shape[0],), W.dtype)])
  out = _build(flat, table)(xf, idxf, W)
  return out.reshape(x.shape)


# speedup vs baseline: 1.0165x; 1.0165x over previous
import functools

import jax
import jax.numpy as jnp
import numpy as np
from jax import lax
from jax.experimental import pallas as pl
from jax.experimental.pallas import tpu as pltpu
from jax.experimental.pallas import tpu_sc as plsc

_L = 16
_BITS = [int(np.uint32(1 << b).astype(np.int32)) for b in range(32)]


@functools.lru_cache(maxsize=None)
def _build(flat: int, table: int):
  mesh = plsc.VectorSubcoreMesh(core_axis_name="c", subcore_axis_name="s")
  nc, ns = mesh.num_cores, mesh.num_subcores
  nw = nc * ns
  per_w = flat // nw
  assert per_w * nw == flat
  chunk = 10240
  assert per_w % chunk == 0
  nchunk = per_w // chunk
  assert nchunk >= 2

  nwords = table // 32
  per_sub_ent = table // ns
  per_sub_w = nwords // ns
  wblk = 8192
  nwblk = per_sub_ent // wblk
  gpb = wblk // (32 * _L)
  assert table % 32 == 0 and nwords % ns == 0 and per_sub_ent % wblk == 0

  def body(x_hbm, idx_hbm, w_hbm, out_hbm, mask_hbm, idx0, idx1, x0, x1,
           o0, o1, mask_v, word_v, is0, is1, os0, os1, msem):
    s = lax.axis_index("s")
    lanes = lax.iota(jnp.int32, _L)
    wid = s * nc + lax.axis_index("c")
    idxb, xb, ob = [idx0, idx1], [x0, x1], [o0, o1]
    isem, osem = [is0, is1], [os0, os1]

    def cbase(k):
      return pl.multiple_of(wid * per_w + k * chunk, chunk)

    ih, oh = {}, {}

    def start_in(k):
      p = k & 1
      b = cbase(k)
      ih[k] = (
          pltpu.async_copy(idx_hbm.at[pl.ds(b, chunk)], idxb[p], isem[p]),
          pltpu.async_copy(x_hbm.at[pl.ds(b, chunk)], xb[p], isem[p]),
      )

    start_in(0)
    start_in(1)

    ent_base = pl.multiple_of(s * per_sub_ent, wblk)
    wh = {}
    for kb in range(min(2, nwblk)):
      wh[kb] = pltpu.async_copy(
          w_hbm.at[pl.ds(ent_base + kb * wblk, wblk)],
          ob[kb].at[pl.ds(0, wblk)], osem[kb])
    for kb in range(nwblk):
      wh.pop(kb).wait()
      w_chunk = ob[kb & 1]

      @plsc.parallel_loop(0, gpb, 1, unroll=2)
      def pack_group(g, kb=kb, w_chunk=w_chunk):
        base = lanes * 32 + g * (32 * _L)
        acc = jnp.zeros((_L,), jnp.int32)
        for b in range(32):
          v = plsc.load_gather(w_chunk, [base + b])
          acc = acc | jnp.where(v > 0.5, jnp.int32(_BITS[b]), jnp.int32(0))
        word_v[pl.ds(pl.multiple_of(kb * gpb * _L + g * _L, _L), _L)] = acc

      if kb + 2 < nwblk:
        wh[kb + 2] = pltpu.async_copy(
            w_hbm.at[pl.ds(ent_base + (kb + 2) * wblk, wblk)],
            ob[kb & 1].at[pl.ds(0, wblk)], osem[kb & 1])

    pltpu.sync_copy(word_v,
                    mask_hbm.at[pl.ds(pl.multiple_of(s * per_sub_w, 8),
                                      per_sub_w)])
    plsc.subcore_barrier()
    pltpu.sync_copy(mask_hbm, mask_v)


    for k in range(nchunk):
      p = k & 1
      ha, hb = ih.pop(k)
      ha.wait()
      hb.wait()
      if k >= 2:
        oh.pop(k - 2).wait()
      iv_ref, xv, ov = idxb[p], xb[p], ob[p]

      @plsc.parallel_loop(0, chunk // _L, 1, unroll=4)
      def _(i):
        sl = pl.ds(pl.multiple_of(i * _L, _L), _L)
        iv = iv_ref[sl]
        words = plsc.load_gather(mask_v, [lax.shift_right_logical(iv, 5)])
        t = lax.shift_left(words, jnp.bitwise_and(jnp.bitwise_not(iv), 31))
        m = lax.shift_right_arithmetic(t, 31)
        ov[sl] = plsc.bitcast(plsc.bitcast(xv[sl], jnp.int32) & m,
                              jnp.float32)

      oh[k] = pltpu.async_copy(ov, out_hbm.at[pl.ds(cbase(k), chunk)],
                               osem[p])
      if k + 2 < nchunk:
        start_in(k + 2)

    oh.pop(nchunk - 2).wait()
    oh.pop(nchunk - 1).wait()

  return pl.kernel(
      body,
      out_type=(jax.ShapeDtypeStruct((flat,), jnp.float32),
                jax.ShapeDtypeStruct((nwords,), jnp.int32)),
      mesh=mesh,
      compiler_params=pltpu.CompilerParams(needs_layout_passes=False),
      scratch_types=[
          pltpu.VMEM((chunk,), jnp.int32),
          pltpu.VMEM((chunk,), jnp.int32),
          pltpu.VMEM((chunk,), jnp.float32),
          pltpu.VMEM((chunk,), jnp.float32),
          pltpu.VMEM((chunk,), jnp.float32),
          pltpu.VMEM((chunk,), jnp.float32),
          pltpu.VMEM((nwords,), jnp.int32),
          pltpu.VMEM((per_sub_w,), jnp.int32),
          pltpu.SemaphoreType.DMA,
          pltpu.SemaphoreType.DMA,
          pltpu.SemaphoreType.DMA,
          pltpu.SemaphoreType.DMA,
          pltpu.SemaphoreType.DMA,
      ],
  )


def kernel(x, idx, W):
  flat = x.size
  xf = x.reshape(flat)
  idxf = idx.reshape(flat).astype(jnp.int32)
  align = 16 * 8192
  table = ((W.shape[0] + align - 1) // align) * align
  if table != W.shape[0]:
    W = jnp.concatenate([W, jnp.zeros((table - W.shape[0],), W.dtype)])
  out, _ = _build(flat, table)(xf, idxf, W)
  return out.reshape(x.shape)


# speedup vs baseline: 1.2153x; 1.1956x over previous
import functools

import jax
import jax.numpy as jnp
import numpy as np
from jax import lax
from jax.experimental import pallas as pl
from jax.experimental.pallas import tpu as pltpu
from jax.experimental.pallas import tpu_sc as plsc

_L = 16
_BITS = [int(np.uint32(1 << b).astype(np.int32)) for b in range(32)]


def _mesh():
  return plsc.VectorSubcoreMesh(core_axis_name="c", subcore_axis_name="s")


@functools.lru_cache(maxsize=None)
def _build_pack(table: int):
  mesh = _mesh()
  nw = mesh.num_cores * mesh.num_subcores
  nwords = table // 32
  per_w_ent = table // nw
  per_w_words = nwords // nw
  wblk = 8192
  nwblk = per_w_ent // wblk
  gpb = wblk // (32 * _L)
  assert table % 32 == 0 and nwords % nw == 0 and per_w_ent % wblk == 0

  def body(w_hbm, mask_hbm, wc0, wc1, word_v, s0, s1):
    wid = lax.axis_index("s") * mesh.num_cores + lax.axis_index("c")
    lanes = lax.iota(jnp.int32, _L)
    ent_base = pl.multiple_of(wid * per_w_ent, wblk)
    wcb, sems = [wc0, wc1], [s0, s1]

    def start(kb):
      return pltpu.async_copy(w_hbm.at[pl.ds(ent_base + kb * wblk, wblk)],
                              wcb[kb & 1], sems[kb & 1])

    wh = {kb: start(kb) for kb in range(min(2, nwblk))}
    for kb in range(nwblk):
      wh.pop(kb).wait()
      w_chunk = wcb[kb & 1]

      @plsc.parallel_loop(0, gpb, 1, unroll=2)
      def pack_group(g, kb=kb, w_chunk=w_chunk):
        base = lanes * 32 + g * (32 * _L)
        acc = jnp.zeros((_L,), jnp.int32)
        for b in range(32):
          v = plsc.load_gather(w_chunk, [base + b])
          acc = acc | jnp.where(v > 0.5, jnp.int32(_BITS[b]), jnp.int32(0))
        word_v[pl.ds(pl.multiple_of(kb * gpb * _L + g * _L, _L), _L)] = acc

      if kb + 2 < nwblk:
        wh[kb + 2] = start(kb + 2)

    pltpu.sync_copy(
        word_v, mask_hbm.at[pl.ds(pl.multiple_of(wid * per_w_words, 8),
                                  per_w_words)])

  return pl.kernel(
      body,
      out_type=jax.ShapeDtypeStruct((nwords,), jnp.int32),
      mesh=mesh,
      compiler_params=pltpu.CompilerParams(needs_layout_passes=False),
      scratch_types=[
          pltpu.VMEM((wblk,), jnp.float32),
          pltpu.VMEM((wblk,), jnp.float32),
          pltpu.VMEM((per_w_words,), jnp.int32),
          pltpu.SemaphoreType.DMA,
          pltpu.SemaphoreType.DMA,
      ],
  )


@functools.lru_cache(maxsize=None)
def _build_apply(flat: int, nwords: int):
  mesh = _mesh()
  nw = mesh.num_cores * mesh.num_subcores
  per_w = flat // nw
  assert per_w * nw == flat
  chunk = 10240
  assert per_w % chunk == 0
  nchunk = per_w // chunk
  assert nchunk >= 2

  def body(x_hbm, idx_hbm, mask_hbm, out_hbm, idx0, idx1, x0, x1, o0, o1,
           mask_v, is0, is1, os0, os1, msem):
    wid = lax.axis_index("s") * mesh.num_cores + lax.axis_index("c")
    idxb, xb, ob = [idx0, idx1], [x0, x1], [o0, o1]
    isem, osem = [is0, is1], [os0, os1]

    def cbase(k):
      return pl.multiple_of(wid * per_w + k * chunk, chunk)

    ih, oh = {}, {}

    def start_in(k):
      p = k & 1
      b = cbase(k)
      ih[k] = (
          pltpu.async_copy(idx_hbm.at[pl.ds(b, chunk)], idxb[p], isem[p]),
          pltpu.async_copy(x_hbm.at[pl.ds(b, chunk)], xb[p], isem[p]),
      )

    start_in(0)
    mh = pltpu.async_copy(mask_hbm, mask_v, msem)
    start_in(1)
    mh.wait()

    for k in range(nchunk):
      p = k & 1
      ha, hb = ih.pop(k)
      ha.wait()
      hb.wait()
      if k >= 2:
        oh.pop(k - 2).wait()
      iv_ref, xv, ov = idxb[p], xb[p], ob[p]

      @plsc.parallel_loop(0, chunk // _L, 1, unroll=4)
      def _(i):
        sl = pl.ds(pl.multiple_of(i * _L, _L), _L)
        iv = iv_ref[sl]
        words = plsc.load_gather(mask_v, [lax.shift_right_logical(iv, 5)])
        t = lax.shift_left(words, jnp.bitwise_and(jnp.bitwise_not(iv), 31))
        m = lax.shift_right_arithmetic(t, 31)
        ov[sl] = plsc.bitcast(plsc.bitcast(xv[sl], jnp.int32) & m,
                              jnp.float32)

      oh[k] = pltpu.async_copy(ov, out_hbm.at[pl.ds(cbase(k), chunk)],
                               osem[p])
      if k + 2 < nchunk:
        start_in(k + 2)

    oh.pop(nchunk - 2).wait()
    oh.pop(nchunk - 1).wait()

  return pl.kernel(
      body,
      out_type=jax.ShapeDtypeStruct((flat,), jnp.float32),
      mesh=mesh,
      compiler_params=pltpu.CompilerParams(needs_layout_passes=False),
      scratch_types=[
          pltpu.VMEM((chunk,), jnp.int32),
          pltpu.VMEM((chunk,), jnp.int32),
          pltpu.VMEM((chunk,), jnp.float32),
          pltpu.VMEM((chunk,), jnp.float32),
          pltpu.VMEM((chunk,), jnp.float32),
          pltpu.VMEM((chunk,), jnp.float32),
          pltpu.VMEM((nwords,), jnp.int32),
          pltpu.SemaphoreType.DMA,
          pltpu.SemaphoreType.DMA,
          pltpu.SemaphoreType.DMA,
          pltpu.SemaphoreType.DMA,
          pltpu.SemaphoreType.DMA,
      ],
  )


def kernel(x, idx, W):
  flat = x.size
  xf = x.reshape(flat)
  idxf = idx.reshape(flat).astype(jnp.int32)
  align = 32 * 8192
  table = ((W.shape[0] + align - 1) // align) * align
  if table != W.shape[0]:
    W = jnp.concatenate([W, jnp.zeros((table - W.shape[0],), W.dtype)])
  mask = _build_pack(table)(W)
  out = _build_apply(flat, table // 32)(xf, idxf, mask)
  return out.reshape(x.shape)


# speedup vs baseline: 1.2191x; 1.0031x over previous
import functools

import jax
import jax.numpy as jnp
import numpy as np
from jax import lax
from jax.experimental import pallas as pl
from jax.experimental.pallas import tpu as pltpu
from jax.experimental.pallas import tpu_sc as plsc

_L = 16
_BITS = [int(np.uint32(1 << b).astype(np.int32)) for b in range(32)]


def _mesh():
  return plsc.VectorSubcoreMesh(core_axis_name="c", subcore_axis_name="s")


@functools.lru_cache(maxsize=None)
def _build_pack(table: int):
  mesh = _mesh()
  nw = mesh.num_cores * mesh.num_subcores
  nwords = table // 32
  per_w_ent = table // nw
  per_w_words = nwords // nw
  wblk = 8192
  nwblk = per_w_ent // wblk
  gpb = wblk // (32 * _L)
  assert table % 32 == 0 and nwords % nw == 0 and per_w_ent % wblk == 0

  def body(w_hbm, mask_hbm, wc0, wc1, word_v, s0, s1):
    wid = lax.axis_index("s") * mesh.num_cores + lax.axis_index("c")
    lanes = lax.iota(jnp.int32, _L)
    ent_base = pl.multiple_of(wid * per_w_ent, wblk)
    wcb, sems = [wc0, wc1], [s0, s1]

    def start(kb):
      return pltpu.async_copy(w_hbm.at[pl.ds(ent_base + kb * wblk, wblk)],
                              wcb[kb & 1], sems[kb & 1])

    wh = {kb: start(kb) for kb in range(min(2, nwblk))}
    for kb in range(nwblk):
      wh.pop(kb).wait()
      w_chunk = wcb[kb & 1]

      @plsc.parallel_loop(0, gpb, 1, unroll=2)
      def pack_group(g, kb=kb, w_chunk=w_chunk):
        base = lanes * 32 + g * (32 * _L)
        acc = jnp.zeros((_L,), jnp.int32)
        for b in range(32):
          v = plsc.load_gather(w_chunk, [base + b])
          acc = acc | jnp.where(v > 0.5, jnp.int32(_BITS[b]), jnp.int32(0))
        word_v[pl.ds(pl.multiple_of(kb * gpb * _L + g * _L, _L), _L)] = acc

      if kb + 2 < nwblk:
        wh[kb + 2] = start(kb + 2)

    pltpu.sync_copy(
        word_v, mask_hbm.at[pl.ds(pl.multiple_of(wid * per_w_words, 8),
                                  per_w_words)])

  return pl.kernel(
      body,
      out_type=jax.ShapeDtypeStruct((nwords,), jnp.int32),
      mesh=mesh,
      compiler_params=pltpu.CompilerParams(needs_layout_passes=False),
      scratch_types=[
          pltpu.VMEM((wblk,), jnp.float32),
          pltpu.VMEM((wblk,), jnp.float32),
          pltpu.VMEM((per_w_words,), jnp.int32),
          pltpu.SemaphoreType.DMA,
          pltpu.SemaphoreType.DMA,
      ],
  )


@functools.lru_cache(maxsize=None)
def _build_apply(flat: int, nwords: int):
  mesh = _mesh()
  nw = mesh.num_cores * mesh.num_subcores
  per_w = flat // nw
  assert per_w * nw == flat
  chunk = 12800
  assert per_w % chunk == 0
  nchunk = per_w // chunk
  assert nchunk >= 2

  def body(x_hbm, idx_hbm, mask_hbm, out_hbm, idx0, idx1, x0, x1, o0, o1,
           mask_v, is0, is1, os0, os1, msem):
    wid = lax.axis_index("s") * mesh.num_cores + lax.axis_index("c")
    idxb, xb, ob = [idx0, idx1], [x0, x1], [o0, o1]
    isem, osem = [is0, is1], [os0, os1]

    def cbase(k):
      return pl.multiple_of(wid * per_w + k * chunk, chunk)

    ih, oh = {}, {}

    def start_in(k):
      p = k & 1
      b = cbase(k)
      ih[k] = (
          pltpu.async_copy(idx_hbm.at[pl.ds(b, chunk)], idxb[p], isem[p]),
          pltpu.async_copy(x_hbm.at[pl.ds(b, chunk)], xb[p], isem[p]),
      )

    start_in(0)
    mh = pltpu.async_copy(mask_hbm, mask_v, msem)
    start_in(1)
    mh.wait()

    for k in range(nchunk):
      p = k & 1
      ha, hb = ih.pop(k)
      ha.wait()
      hb.wait()
      if k >= 2:
        oh.pop(k - 2).wait()
      iv_ref, xv, ov = idxb[p], xb[p], ob[p]

      @plsc.parallel_loop(0, chunk // _L, 1, unroll=8)
      def _(i):
        sl = pl.ds(pl.multiple_of(i * _L, _L), _L)
        iv = iv_ref[sl]
        words = plsc.load_gather(mask_v, [lax.shift_right_logical(iv, 5)])
        t = lax.shift_left(words, jnp.bitwise_and(jnp.bitwise_not(iv), 31))
        m = lax.shift_right_arithmetic(t, 31)
        ov[sl] = plsc.bitcast(plsc.bitcast(xv[sl], jnp.int32) & m,
                              jnp.float32)

      oh[k] = pltpu.async_copy(ov, out_hbm.at[pl.ds(cbase(k), chunk)],
                               osem[p])
      if k + 2 < nchunk:
        start_in(k + 2)

    oh.pop(nchunk - 2).wait()
    oh.pop(nchunk - 1).wait()

  return pl.kernel(
      body,
      out_type=jax.ShapeDtypeStruct((flat,), jnp.float32),
      mesh=mesh,
      compiler_params=pltpu.CompilerParams(needs_layout_passes=False),
      scratch_types=[
          pltpu.VMEM((chunk,), jnp.int32),
          pltpu.VMEM((chunk,), jnp.int32),
          pltpu.VMEM((chunk,), jnp.float32),
          pltpu.VMEM((chunk,), jnp.float32),
          pltpu.VMEM((chunk,), jnp.float32),
          pltpu.VMEM((chunk,), jnp.float32),
          pltpu.VMEM((nwords,), jnp.int32),
          pltpu.SemaphoreType.DMA,
          pltpu.SemaphoreType.DMA,
          pltpu.SemaphoreType.DMA,
          pltpu.SemaphoreType.DMA,
          pltpu.SemaphoreType.DMA,
      ],
  )


def kernel(x, idx, W):
  flat = x.size
  xf = x.reshape(flat)
  idxf = idx.reshape(flat).astype(jnp.int32)
  align = 32 * 8192
  table = ((W.shape[0] + align - 1) // align) * align
  if table != W.shape[0]:
    W = jnp.concatenate([W, jnp.zeros((table - W.shape[0],), W.dtype)])
  mask = _build_pack(table)(W)
  out = _build_apply(flat, table // 32)(xf, idxf, mask)
  return out.reshape(x.shape)
